# Initial kernel scaffold; baseline (speedup 1.0000x reference)
import functools

import jax
import jax.numpy as jnp
from jax import lax
from jax.experimental import pallas as pl
from jax.experimental.pallas import tpu as pltpu
from jax.experimental.pallas import tpu_sc as plsc

MAXLEN = 200
EMBED = 64
BATCH = 4096
LANES = 16
EV = EMBED // LANES

_info = plsc.get_sparse_core_info()
NC = _info.num_cores
NS = _info.num_subcores
NW = NC * NS
ROWS_PER_W = BATCH // NW
R = 4
GROUPS = ROWS_PER_W // R


def _emb_body(x_hbm, tok_hbm, pos_hbm, out_hbm, pos_v, idx_v, rows_v, sem):
    wid = lax.axis_index("s") * NC + lax.axis_index("c")
    base_row = wid * ROWS_PER_W

    pltpu.sync_copy(pos_hbm, pos_v)

    def group_body(g, _):
        row0 = base_row + g * R
        pltpu.sync_copy(x_hbm.at[pl.ds(row0, R)], idx_v)
        for r in range(R):
            pltpu.async_copy(tok_hbm.at[idx_v.at[r]], rows_v.at[r], sem).wait()

        def add_row(m, _):
            for r in range(R):
                for c in range(EV):
                    sl = pl.ds(c * LANES, LANES)
                    rows_v[r, m, sl] = rows_v[r, m, sl] + pos_v[m, sl]
            return ()

        lax.fori_loop(0, MAXLEN, add_row, ())
        pltpu.sync_copy(rows_v, out_hbm.at[pl.ds(row0, R)])
        return ()

    lax.fori_loop(0, GROUPS, group_body, ())


def kernel(x, token_table, pos_table):
    mesh = plsc.VectorSubcoreMesh(core_axis_name="c", subcore_axis_name="s")
    f = functools.partial(
        pl.kernel,
        mesh=mesh,
        out_type=jax.ShapeDtypeStruct((BATCH, MAXLEN, EMBED), jnp.float32),
        scratch_types=[
            pltpu.VMEM((MAXLEN, EMBED), jnp.float32),
            pltpu.VMEM((R, MAXLEN), jnp.int32),
            pltpu.VMEM((R, MAXLEN, EMBED), jnp.float32),
            pltpu.SemaphoreType.DMA,
        ],
    )(_emb_body)
    return f(x.astype(jnp.int32), token_table, pos_table)


# speedup vs baseline: 3.3421x; 3.3421x over previous
import functools

import jax
import jax.numpy as jnp
from jax import lax
from jax.experimental import pallas as pl
from jax.experimental.pallas import tpu as pltpu
from jax.experimental.pallas import tpu_sc as plsc

MAXLEN = 200
EMBED = 64
BATCH = 4096
LANES = 16
EV = EMBED // LANES

_info = plsc.get_sparse_core_info()
NC = _info.num_cores
NS = _info.num_subcores
NW = NC * NS
ROWS_PER_W = BATCH // NW
R = 4
GROUPS = ROWS_PER_W // R


def _emb_body(x_hbm, tok_hbm, pos_hbm, out_hbm, pos_v, idx_v, rows_v, sem):
    wid = lax.axis_index("s") * NC + lax.axis_index("c")
    base_row = wid * ROWS_PER_W

    pltpu.sync_copy(pos_hbm, pos_v)

    def group_body(g, _):
        row0 = base_row + g * R
        pltpu.sync_copy(x_hbm.at[pl.ds(row0, R)], idx_v)
        for r in range(R):
            pltpu.async_copy(tok_hbm.at[idx_v.at[r]], rows_v.at[r], sem).wait()

        def add_row(m, _):
            for r in range(R):
                for c in range(EV):
                    sl = pl.ds(c * LANES, LANES)
                    rows_v[r, m, sl] = rows_v[r, m, sl] + pos_v[m, sl]
            return ()

        lax.fori_loop(0, MAXLEN, add_row, ())
        pltpu.sync_copy(rows_v, out_hbm.at[pl.ds(row0, R)])
        return ()

    lax.fori_loop(0, GROUPS, group_body, ())


def kernel(x, token_table, pos_table):
    mesh = plsc.VectorSubcoreMesh(core_axis_name="c", subcore_axis_name="s")
    f = functools.partial(
        pl.kernel,
        mesh=mesh,
        compiler_params=pltpu.CompilerParams(use_tc_tiling_on_sc=False),
        out_type=jax.ShapeDtypeStruct((BATCH, MAXLEN, EMBED), jnp.float32),
        scratch_types=[
            pltpu.VMEM((MAXLEN, EMBED), jnp.float32),
            pltpu.VMEM((R, MAXLEN), jnp.int32),
            pltpu.VMEM((R, MAXLEN, EMBED), jnp.float32),
            pltpu.SemaphoreType.DMA,
        ],
    )(_emb_body)
    return f(x.astype(jnp.int32), token_table, pos_table)


# speedup vs baseline: 4.0179x; 1.2022x over previous
import functools

import jax
import jax.numpy as jnp
from jax import lax
from jax.experimental import pallas as pl
from jax.experimental.pallas import tpu as pltpu
from jax.experimental.pallas import tpu_sc as plsc

MAXLEN = 200
EMBED = 64
BATCH = 4096
LANES = 16
EV = EMBED // LANES

_info = plsc.get_sparse_core_info()
NC = _info.num_cores
NS = _info.num_subcores
NW = NC * NS
ROWS_PER_W = BATCH // NW
NBUF = 4
ROUNDS = ROWS_PER_W // NBUF


def _emb_body(x_hbm, tok_hbm, pos_hbm, out_hbm,
              pos_v, idx_all, r0, r1, r2, r3,
              g0, g1, g2, g3, w0, w1, w2, w3):
    rows = (r0, r1, r2, r3)
    gsem = (g0, g1, g2, g3)
    wsem = (w0, w1, w2, w3)

    wid = lax.axis_index("s") * NC + lax.axis_index("c")
    rbase = wid * ROWS_PER_W

    pltpu.sync_copy(pos_hbm, pos_v)
    pltpu.sync_copy(x_hbm.at[pl.ds(rbase, ROWS_PER_W)], idx_all)

    pltpu.async_copy(tok_hbm.at[idx_all.at[0]], rows[0], gsem[0])

    def round_body(rnd, _):
        for p in range(NBUF):
            g = rnd * NBUF + p
            q = (p + 1) % NBUF

            pltpu.make_async_copy(
                tok_hbm.at[pl.ds(0, MAXLEN)], rows[p], gsem[p]).wait()

            def _fire(q=q, g=g):
                pltpu.make_async_copy(
                    rows[q], out_hbm.at[pl.ds(0, MAXLEN)], wsem[q]).wait()
                pltpu.async_copy(
                    tok_hbm.at[idx_all.at[g + 1]], rows[q], gsem[q])

            if p < NBUF - 1:
                def _fire_first(q=q, g=g):
                    pltpu.async_copy(
                        tok_hbm.at[idx_all.at[g + 1]], rows[q], gsem[q])
                pl.when(rnd >= 1)(_fire)
                pl.when(rnd == 0)(_fire_first)
            else:
                pl.when(rnd < ROUNDS - 1)(_fire)

            rp = rows[p]

            @plsc.parallel_loop(0, MAXLEN, 1, unroll=4)
            def _add(m, rp=rp):
                for c in range(EV):
                    sl = pl.ds(c * LANES, LANES)
                    rp[m, sl] = rp[m, sl] + pos_v[m, sl]

            pltpu.async_copy(
                rows[p], out_hbm.at[pl.ds((rbase + g) * MAXLEN, MAXLEN)],
                wsem[p])
        return ()

    lax.fori_loop(0, ROUNDS, round_body, ())

    for p in range(NBUF):
        pltpu.make_async_copy(
            rows[p], out_hbm.at[pl.ds(0, MAXLEN)], wsem[p]).wait()


def kernel(x, token_table, pos_table):
    mesh = plsc.VectorSubcoreMesh(core_axis_name="c", subcore_axis_name="s")
    f = functools.partial(
        pl.kernel,
        mesh=mesh,
        compiler_params=pltpu.CompilerParams(use_tc_tiling_on_sc=False),
        out_type=jax.ShapeDtypeStruct((BATCH * MAXLEN, EMBED), jnp.float32),
        scratch_types=[
            pltpu.VMEM((MAXLEN, EMBED), jnp.float32),
            pltpu.VMEM((ROWS_PER_W, MAXLEN), jnp.int32),
        ] + [pltpu.VMEM((MAXLEN, EMBED), jnp.float32) for _ in range(NBUF)]
          + [pltpu.SemaphoreType.DMA for _ in range(2 * NBUF)],
    )(_emb_body)
    out = f(x.astype(jnp.int32), token_table, pos_table)
    return out.reshape(BATCH, MAXLEN, EMBED)


# speedup vs baseline: 4.0230x; 1.0013x over previous
import functools

import jax
import jax.numpy as jnp
from jax import lax
from jax.experimental import pallas as pl
from jax.experimental.pallas import tpu as pltpu
from jax.experimental.pallas import tpu_sc as plsc

MAXLEN = 200
EMBED = 64
BATCH = 4096
LANES = 16
EV = EMBED // LANES

_info = plsc.get_sparse_core_info()
NC = _info.num_cores
NS = _info.num_subcores
NW = NC * NS
ROWS_PER_W = BATCH // NW
NBUF = 4
ROUNDS = ROWS_PER_W // NBUF


def _emb_body(x_hbm, tok_hbm, pos_hbm, out_hbm,
              pos_v, idx_all, r0, r1, r2, r3,
              g0, g1, g2, g3, w0, w1, w2, w3):
    rows = (r0, r1, r2, r3)
    gsem = (g0, g1, g2, g3)
    wsem = (w0, w1, w2, w3)

    wid = lax.axis_index("s") * NC + lax.axis_index("c")
    rbase = wid * ROWS_PER_W

    pltpu.sync_copy(pos_hbm, pos_v)
    pltpu.sync_copy(x_hbm.at[pl.ds(rbase, ROWS_PER_W)], idx_all)

    pltpu.async_copy(tok_hbm.at[idx_all.at[0]], rows[0], gsem[0])

    def round_body(rnd, _):
        for p in range(NBUF):
            g = rnd * NBUF + p
            q = (p + 1) % NBUF

            pltpu.make_async_copy(
                tok_hbm.at[pl.ds(0, MAXLEN)], rows[p], gsem[p]).wait()

            def _fire(q=q, g=g):
                pltpu.make_async_copy(
                    rows[q], out_hbm.at[0], wsem[q]).wait()
                pltpu.async_copy(
                    tok_hbm.at[idx_all.at[g + 1]], rows[q], gsem[q])

            if p < NBUF - 1:
                def _fire_first(q=q, g=g):
                    pltpu.async_copy(
                        tok_hbm.at[idx_all.at[g + 1]], rows[q], gsem[q])
                pl.when(rnd >= 1)(_fire)
                pl.when(rnd == 0)(_fire_first)
            else:
                pl.when(rnd < ROUNDS - 1)(_fire)

            rp = rows[p]

            @plsc.parallel_loop(0, MAXLEN, 1, unroll=4)
            def _add(m, rp=rp):
                for c in range(EV):
                    sl = pl.ds(c * LANES, LANES)
                    rp[m, sl] = rp[m, sl] + pos_v[m, sl]

            pltpu.async_copy(rows[p], out_hbm.at[rbase + g], wsem[p])
        return ()

    lax.fori_loop(0, ROUNDS, round_body, ())

    for p in range(NBUF):
        pltpu.make_async_copy(rows[p], out_hbm.at[0], wsem[p]).wait()


def kernel(x, token_table, pos_table):
    mesh = plsc.VectorSubcoreMesh(core_axis_name="c", subcore_axis_name="s")
    f = functools.partial(
        pl.kernel,
        mesh=mesh,
        compiler_params=pltpu.CompilerParams(use_tc_tiling_on_sc=False),
        out_type=jax.ShapeDtypeStruct((BATCH, MAXLEN, EMBED), jnp.float32),
        scratch_types=[
            pltpu.VMEM((MAXLEN, EMBED), jnp.float32),
            pltpu.VMEM((ROWS_PER_W, MAXLEN), jnp.int32),
        ] + [pltpu.VMEM((MAXLEN, EMBED), jnp.float32) for _ in range(NBUF)]
          + [pltpu.SemaphoreType.DMA for _ in range(2 * NBUF)],
    )(_emb_body)
    return f(x.astype(jnp.int32), token_table, pos_table)
